# baseline (speedup 1.0000x reference)
import functools

import numpy as np
import jax
import jax.numpy as jnp
from jax import lax
from jax.experimental import pallas as pl
from jax.experimental.pallas import tpu as pltpu
from jax.experimental.pallas import tpu_sc as plsc

_BN_DIV = np.sqrt(1.0 + 1e-5)
_F32 = jnp.float32


def _fps_body(xa_ref, xb_ref, na_ref, nb_ref, da_ref, db_ref, *, n, npoint):
    n8 = n // 8
    xa = xa_ref[0]
    ya = xa_ref[1]
    za = xa_ref[2]
    xb = xb_ref[0]
    yb = xb_ref[1]
    zb = xb_ref[2]
    rowi = lax.broadcasted_iota(jnp.int32, (8, n8), 0)
    coli = lax.broadcasted_iota(jnp.int32, (8, n8), 1)
    flat = rowi * n8 + coli
    da_ref[...] = jnp.full((8, n8), 1e10, _F32)
    db_ref[...] = jnp.full((8, n8), 1e10, _F32)

    def body(i, fs):
        fa, fb = fs
        zero = _F32(0.0)
        sela = flat == fa
        selb = flat == fb
        cxa = jnp.sum(jnp.where(sela, xa, zero))
        cya = jnp.sum(jnp.where(sela, ya, zero))
        cza = jnp.sum(jnp.where(sela, za, zero))
        cxb = jnp.sum(jnp.where(selb, xb, zero))
        cyb = jnp.sum(jnp.where(selb, yb, zero))
        czb = jnp.sum(jnp.where(selb, zb, zero))
        dxa = xa - cxa
        dya = ya - cya
        dza = za - cza
        dxb = xb - cxb
        dyb = yb - cyb
        dzb = zb - czb
        dista = (dxa * dxa + dza * dza) + dya * dya
        distb = (dxb * dxb + dzb * dzb) + dyb * dyb
        da = jnp.minimum(da_ref[...], dista)
        db = jnp.minimum(db_ref[...], distb)
        da_ref[...] = da
        db_ref[...] = db
        ma = jnp.max(da)
        mb = jnp.max(db)
        fna = jnp.min(jnp.where(da == ma, flat, jnp.int32(n)))
        fnb = jnp.min(jnp.where(db == mb, flat, jnp.int32(n)))
        na_ref[pl.ds(i, 1), :] = jnp.stack([cxa, cya, cza]).reshape(1, 3)
        nb_ref[pl.ds(i, 1), :] = jnp.stack([cxb, cyb, czb]).reshape(1, 3)
        return (fna, fnb)

    lax.fori_loop(0, npoint, body, (jnp.int32(0), jnp.int32(0)))


def _fps2(xa3, xb3, npoint):
    n = xa3.shape[1]
    ra = xa3.reshape(3, 8, n // 8)
    rb = xb3.reshape(3, 8, n // 8)
    return pl.pallas_call(
        functools.partial(_fps_body, n=n, npoint=npoint),
        out_shape=[
            jax.ShapeDtypeStruct((npoint, 3), _F32),
            jax.ShapeDtypeStruct((npoint, 3), _F32),
        ],
        scratch_shapes=[pltpu.VMEM((8, n // 8), _F32),
                        pltpu.VMEM((8, n // 8), _F32)],
    )(ra, rb)


def _ballq_body(new_ref, xyz_ref, idx_ref, *, n, r2, ns):
    ts = new_ref.shape[0]
    c = new_ref[...]
    x3 = xyz_ref[...]
    x = x3[0:1, :]
    y = x3[1:2, :]
    z = x3[2:3, :]
    cx = c[:, 0:1]
    cy = c[:, 1:2]
    cz = c[:, 2:3]
    ssx = (x * x + z * z) + y * y
    ssc = (cx * cx + cz * cz) + cy * cy
    dot = lax.dot_general(c, x3, (((1,), (0,)), ((), ())))
    d2 = ssc + ssx - 2.0 * dot
    col = lax.broadcasted_iota(jnp.int32, (ts, n), 1)
    key = jnp.where(d2 <= r2, col, jnp.int32(n))
    prev = jnp.full((ts, 1), -1, jnp.int32)
    first = None
    for k in range(ns):
        cand = jnp.where(key > prev, key, jnp.int32(n))
        mk = jnp.min(cand, axis=1, keepdims=True)
        if k == 0:
            first = mk
            outk = mk
        else:
            outk = jnp.where(mk == n, first, mk)
        idx_ref[:, k:k + 1] = jnp.minimum(outk, n - 1)
        prev = mk


def _ballq(new_r, xyz3, r2, ns, ts):
    s = new_r.shape[0]
    n = xyz3.shape[1]
    return pl.pallas_call(
        functools.partial(_ballq_body, n=n, r2=r2, ns=ns),
        grid=(s // ts,),
        in_specs=[
            pl.BlockSpec((ts, 3), lambda t: (t, 0)),
            pl.BlockSpec((3, n), lambda t: (0, 0)),
        ],
        out_specs=pl.BlockSpec((ts, ns), lambda t: (t, 0)),
        out_shape=jax.ShapeDtypeStruct((s, ns), jnp.int32),
    )(new_r, xyz3)


def _knn_body(cent_ref, cand_ref, val_ref, idx_ref, *, m, k):
    ts = cent_ref.shape[0]
    c = cent_ref[...]
    x3 = cand_ref[...]
    x = x3[0:1, :]
    y = x3[1:2, :]
    z = x3[2:3, :]
    cx = c[:, 0:1]
    cy = c[:, 1:2]
    cz = c[:, 2:3]
    ssx = (x * x + z * z) + y * y
    ssc = (cx * cx + cz * cz) + cy * cy
    dot = lax.dot_general(c, x3, (((1,), (0,)), ((), ())))
    cur = ssc + ssx - 2.0 * dot
    col = lax.broadcasted_iota(jnp.int32, (ts, m), 1)
    big = _F32(3.0e38)
    for kk in range(k):
        mv = jnp.min(cur, axis=1, keepdims=True)
        jk = jnp.min(jnp.where(cur == mv, col, jnp.int32(m)), axis=1,
                     keepdims=True)
        val_ref[:, kk:kk + 1] = mv
        idx_ref[:, kk:kk + 1] = jk
        cur = jnp.where(col == jk, big, cur)


def _knn(cent_r, cand3, k, ts):
    s = cent_r.shape[0]
    m = cand3.shape[1]
    return pl.pallas_call(
        functools.partial(_knn_body, m=m, k=k),
        grid=(s // ts,),
        in_specs=[
            pl.BlockSpec((ts, 3), lambda t: (t, 0)),
            pl.BlockSpec((3, m), lambda t: (0, 0)),
        ],
        out_specs=[
            pl.BlockSpec((ts, k), lambda t: (t, 0)),
            pl.BlockSpec((ts, k), lambda t: (t, 0)),
        ],
        out_shape=[
            jax.ShapeDtypeStruct((s, k), _F32),
            jax.ShapeDtypeStruct((s, k), jnp.int32),
        ],
    )(cent_r, cand3)


def _sc_gather(table, idx):
    b = idx.shape[0]
    d = table.shape[1]
    nw = 32
    bpw = b // nw
    mesh = plsc.VectorSubcoreMesh(core_axis_name="c", subcore_axis_name="s")

    @functools.partial(
        pl.kernel,
        mesh=mesh,
        compiler_params=pltpu.CompilerParams(use_tc_tiling_on_sc=False),
        out_type=jax.ShapeDtypeStruct((b, d), _F32),
        scratch_types=[
            pltpu.VMEM((bpw,), jnp.int32),
            pltpu.VMEM((bpw, d), _F32),
            pltpu.SemaphoreType.DMA,
        ],
    )
    def k(table_hbm, idx_hbm, out_hbm, idx_v, rows_v, sem):
        wid = lax.axis_index("s") * 2 + lax.axis_index("c")
        base = wid * bpw
        pltpu.sync_copy(idx_hbm.at[pl.ds(base, bpw)], idx_v)
        pltpu.async_copy(table_hbm.at[idx_v], rows_v, sem).wait()
        pltpu.sync_copy(rows_v, out_hbm.at[pl.ds(base, bpw)])

    return k(table, idx)


def _sa_body(g_ref, c_ref, w1_ref, w2_ref, o_ref, *, k, dp, d2):
    ts = c_ref.shape[0]
    c = c_ref[...]
    sub = jnp.concatenate([c, jnp.zeros((ts, dp - 3), _F32)], axis=1)
    h = g_ref[...] - sub[None, :, :]
    a = jnp.maximum(
        lax.dot_general(h.reshape(k * ts, dp), w1_ref[...],
                        (((1,), (1,)), ((), ()))) / _BN_DIV, 0.0)
    b = jnp.maximum(
        lax.dot_general(a, w2_ref[...],
                        (((1,), (1,)), ((), ()))) / _BN_DIV, 0.0)
    o_ref[...] = jnp.max(b.reshape(k, ts, d2), axis=0)


def _sa_mlp(g3, cent_r, w1p, w2, ts):
    k, s, dp = g3.shape
    d2 = w2.shape[0]
    return pl.pallas_call(
        functools.partial(_sa_body, k=k, dp=dp, d2=d2),
        grid=(s // ts,),
        in_specs=[
            pl.BlockSpec((k, ts, dp), lambda t: (0, t, 0)),
            pl.BlockSpec((ts, 3), lambda t: (t, 0)),
            pl.BlockSpec(w1p.shape, lambda t: (0, 0)),
            pl.BlockSpec(w2.shape, lambda t: (0, 0)),
        ],
        out_specs=pl.BlockSpec((ts, d2), lambda t: (t, 0)),
        out_shape=jax.ShapeDtypeStruct((s, d2), _F32),
    )(g3, cent_r, w1p, w2)


def _upconv_body(g_ref, c_ref, f1_ref, wg_ref, wf_ref, o_ref, *, k, dp, df):
    ts = c_ref.shape[0]
    c = c_ref[...]
    sub = jnp.concatenate(
        [jnp.zeros((ts, df), _F32), c, jnp.zeros((ts, dp - df - 3), _F32)],
        axis=1)
    h = g_ref[...] - sub[None, :, :]
    a = jnp.maximum(
        lax.dot_general(h.reshape(k * ts, dp), wg_ref[...],
                        (((1,), (1,)), ((), ()))) / _BN_DIV, 0.0)
    hm = jnp.max(a.reshape(k, ts, 32), axis=0)
    h2 = jnp.concatenate([hm, f1_ref[...]], axis=1)
    o = lax.dot_general(h2, wf_ref[...], (((1,), (1,)), ((), ())))
    o_ref[...] = jnp.maximum(o / _BN_DIV, 0.0)


def _upconv(g3, cent_r, f1, wgp, wf, ts):
    k, s, dp = g3.shape
    df = 64
    return pl.pallas_call(
        functools.partial(_upconv_body, k=k, dp=dp, df=df),
        grid=(s // ts,),
        in_specs=[
            pl.BlockSpec((k, ts, dp), lambda t: (0, t, 0)),
            pl.BlockSpec((ts, 3), lambda t: (t, 0)),
            pl.BlockSpec((ts, 32), lambda t: (t, 0)),
            pl.BlockSpec(wgp.shape, lambda t: (0, 0)),
            pl.BlockSpec(wf.shape, lambda t: (0, 0)),
        ],
        out_specs=pl.BlockSpec((ts, 32), lambda t: (t, 0)),
        out_shape=jax.ShapeDtypeStruct((s, 32), _F32),
    )(g3, cent_r, f1, wgp, wf)


def _fp_body(g_ref, w_ref, fea_ref, wp_ref, cw_ref, cb_ref, o_ref, *, dpad):
    ts = w_ref.shape[0]
    g = g_ref[...]
    w = w_ref[...]
    t0 = w[:, 0:1] * g[0]
    t1 = w[:, 1:2] * g[1]
    t2 = w[:, 2:3] * g[2]
    interp = (t0 + t1) + t2
    hcat = jnp.concatenate(
        [interp, fea_ref[...], jnp.zeros((ts, dpad), _F32)], axis=1)
    l0 = jnp.maximum(
        lax.dot_general(hcat, wp_ref[...], (((1,), (1,)), ((), ()))) / _BN_DIV,
        0.0)
    xv = jnp.maximum(l0 / _BN_DIV, 0.0)
    f = lax.dot_general(xv, cw_ref[...], (((1,), (1,)), ((), ()))) + cb_ref[...]
    o_ref[...] = f


def _fp_final(g3, w3, fea, wpp, cw, cb, ts):
    _, s, d = g3.shape
    do = cw.shape[0]
    cb2 = cb.reshape(1, do)
    dpad = wpp.shape[1] - d - fea.shape[1]
    return pl.pallas_call(
        functools.partial(_fp_body, dpad=dpad),
        grid=(s // ts,),
        in_specs=[
            pl.BlockSpec((3, ts, d), lambda t: (0, t, 0)),
            pl.BlockSpec((ts, 3), lambda t: (t, 0)),
            pl.BlockSpec((ts, fea.shape[1]), lambda t: (t, 0)),
            pl.BlockSpec(wpp.shape, lambda t: (0, 0)),
            pl.BlockSpec(cw.shape, lambda t: (0, 0)),
            pl.BlockSpec((1, do), lambda t: (0, 0)),
        ],
        out_specs=pl.BlockSpec((ts, do), lambda t: (t, 0)),
        out_shape=jax.ShapeDtypeStruct((s, do), _F32),
    )(g3, w3, fea, wpp, cw, cb2)


def _padw(w, cols):
    o, c = w.shape
    return jnp.concatenate([w, jnp.zeros((o, cols - c), _F32)], axis=1)


def _sa1_stage(pc, fea, new1r, prm):
    bq1 = _ballq(new1r, pc.T, 1.0, 32, ts=512)
    t1 = jnp.concatenate([pc, fea, jnp.zeros((8192, 3), _F32)], axis=1)
    g1 = _sc_gather(t1, bq1.T.reshape(-1))
    return _sa_mlp(g1.reshape(32, 4096, 16), new1r,
                   _padw(prm['sa1_w1'], 16), prm['sa1_w2'], ts=512)


def _tail_stage(pc, fea, new1r, new2r, l1_f, prm):
    bq2 = _ballq(new2r, new1r.T, 4.0, 32, ts=512)
    t2 = jnp.concatenate([new1r, l1_f, jnp.zeros((4096, 13), _F32)], axis=1)
    g2 = _sc_gather(t2, bq2.T.reshape(-1))
    l2_f = _sa_mlp(g2.reshape(32, 1024, 48), new2r,
                   _padw(prm['sa2_w1'], 48), prm['sa2_w2'], ts=256)

    _, su_idx = _knn(new1r, new2r.T, 8, ts=512)
    tg = jnp.concatenate([l2_f, new2r, jnp.zeros((1024, 13), _F32)], axis=1)
    gg = _sc_gather(tg, su_idx.T.reshape(-1))
    l1_fnew = _upconv(gg.reshape(8, 4096, 80), new1r, l1_f,
                      _padw(prm['su1_wg'], 80), prm['su1_wf'], ts=512)

    d3, fp_idx = _knn(pc, new1r.T, 3, ts=512)
    dist_recip = 1.0 / (d3 + 1e-8)
    w3 = dist_recip / jnp.sum(dist_recip, axis=-1, keepdims=True)
    gf = _sc_gather(l1_fnew, fp_idx.T.reshape(-1))
    f = _fp_final(gf.reshape(3, 8192, 32), w3, fea, _padw(prm['fp_w'], 48),
                  prm['conv2_w'], prm['conv2_b'], ts=1024)
    return jnp.concatenate([pc, f], axis=-1)


def kernel(pc1, fea1, weights1, pc2, fea2, weights2, params):
    pa, fa = pc1[0], fea1[0]
    pb, fb = pc2[0], fea2[0]
    new1a, new1b = _fps2(pa.T, pb.T, 4096)
    l1fa = _sa1_stage(pa, fa, new1a, params)
    l1fb = _sa1_stage(pb, fb, new1b, params)
    new2a, new2b = _fps2(new1a.T, new1b.T, 1024)
    sf = _tail_stage(pa, fa, new1a, new2a, l1fa, params)
    tf = _tail_stage(pb, fb, new1b, new2b, l1fb, params)
    return (sf[None], tf[None])


# speedup vs baseline: 1.1682x; 1.1682x over previous
import functools

import numpy as np
import jax
import jax.numpy as jnp
from jax import lax
from jax.experimental import pallas as pl
from jax.experimental.pallas import tpu as pltpu
from jax.experimental.pallas import tpu_sc as plsc

_BN_DIV = np.sqrt(1.0 + 1e-5)
_F32 = jnp.float32


def _fps_body(xa_ref, xb_ref, sa_ref, sb_ref, na_ref, nb_ref, da_ref, db_ref,
              *, n, npoint):
    n8 = n // 8
    xa = xa_ref[0]
    ya = xa_ref[1]
    za = xa_ref[2]
    xb = xb_ref[0]
    yb = xb_ref[1]
    zb = xb_ref[2]
    rowi = lax.broadcasted_iota(jnp.int32, (8, n8), 0)
    coli = lax.broadcasted_iota(jnp.int32, (8, n8), 1)
    flat = rowi * n8 + coli
    da_ref[...] = jnp.full((8, n8), 1e10, _F32)
    db_ref[...] = jnp.full((8, n8), 1e10, _F32)

    def body(i, fs):
        fa, fb = fs
        fa3 = fa * 3
        fb3 = fb * 3
        cxa = sa_ref[fa3]
        cya = sa_ref[fa3 + 1]
        cza = sa_ref[fa3 + 2]
        cxb = sb_ref[fb3]
        cyb = sb_ref[fb3 + 1]
        czb = sb_ref[fb3 + 2]
        dxa = xa - cxa
        dya = ya - cya
        dza = za - cza
        dxb = xb - cxb
        dyb = yb - cyb
        dzb = zb - czb
        dista = (dxa * dxa + dza * dza) + dya * dya
        distb = (dxb * dxb + dzb * dzb) + dyb * dyb
        da = jnp.minimum(da_ref[...], dista)
        db = jnp.minimum(db_ref[...], distb)
        da_ref[...] = da
        db_ref[...] = db
        ma = jnp.max(da)
        mb = jnp.max(db)
        fna = jnp.min(jnp.where(da == ma, flat, jnp.int32(n)))
        fnb = jnp.min(jnp.where(db == mb, flat, jnp.int32(n)))
        na_ref[pl.ds(i, 1), :] = jnp.stack([cxa, cya, cza]).reshape(1, 3)
        nb_ref[pl.ds(i, 1), :] = jnp.stack([cxb, cyb, czb]).reshape(1, 3)
        return (fna, fnb)

    lax.fori_loop(0, npoint, body, (jnp.int32(0), jnp.int32(0)))


def _fps2(xa3, xb3, xar, xbr, npoint):
    n = xa3.shape[1]
    ra = xa3.reshape(3, 8, n // 8)
    rb = xb3.reshape(3, 8, n // 8)
    return pl.pallas_call(
        functools.partial(_fps_body, n=n, npoint=npoint),
        in_specs=[
            pl.BlockSpec((3, 8, n // 8), lambda: (0, 0, 0)),
            pl.BlockSpec((3, 8, n // 8), lambda: (0, 0, 0)),
            pl.BlockSpec(memory_space=pltpu.SMEM),
            pl.BlockSpec(memory_space=pltpu.SMEM),
        ],
        out_shape=[
            jax.ShapeDtypeStruct((npoint, 3), _F32),
            jax.ShapeDtypeStruct((npoint, 3), _F32),
        ],
        scratch_shapes=[pltpu.VMEM((8, n // 8), _F32),
                        pltpu.VMEM((8, n // 8), _F32)],
    )(ra, rb, xar.reshape(-1), xbr.reshape(-1))


def _ballq_body(new_ref, xyz_ref, idx_ref, *, n, r2, ns):
    ts = new_ref.shape[0]
    c = new_ref[...]
    x3 = xyz_ref[...]
    x = x3[0:1, :]
    y = x3[1:2, :]
    z = x3[2:3, :]
    cx = c[:, 0:1]
    cy = c[:, 1:2]
    cz = c[:, 2:3]
    ssx = (x * x + z * z) + y * y
    ssc = (cx * cx + cz * cz) + cy * cy
    dot = lax.dot_general(c, x3, (((1,), (0,)), ((), ())))
    d2 = ssc + ssx - 2.0 * dot
    col = lax.broadcasted_iota(jnp.int32, (ts, n), 1)
    key = jnp.where(d2 <= r2, col, jnp.int32(n))
    prev = jnp.full((ts, 1), -1, jnp.int32)
    first = None
    for k in range(ns):
        cand = jnp.where(key > prev, key, jnp.int32(n))
        mk = jnp.min(cand, axis=1, keepdims=True)
        if k == 0:
            first = mk
            outk = mk
        else:
            outk = jnp.where(mk == n, first, mk)
        idx_ref[:, k:k + 1] = jnp.minimum(outk, n - 1)
        prev = mk


def _ballq(new_r, xyz3, r2, ns, ts):
    s = new_r.shape[0]
    n = xyz3.shape[1]
    return pl.pallas_call(
        functools.partial(_ballq_body, n=n, r2=r2, ns=ns),
        grid=(s // ts,),
        in_specs=[
            pl.BlockSpec((ts, 3), lambda t: (t, 0)),
            pl.BlockSpec((3, n), lambda t: (0, 0)),
        ],
        out_specs=pl.BlockSpec((ts, ns), lambda t: (t, 0)),
        out_shape=jax.ShapeDtypeStruct((s, ns), jnp.int32),
    )(new_r, xyz3)


def _knn_body(cent_ref, cand_ref, val_ref, idx_ref, *, m, k):
    ts = cent_ref.shape[0]
    c = cent_ref[...]
    x3 = cand_ref[...]
    x = x3[0:1, :]
    y = x3[1:2, :]
    z = x3[2:3, :]
    cx = c[:, 0:1]
    cy = c[:, 1:2]
    cz = c[:, 2:3]
    ssx = (x * x + z * z) + y * y
    ssc = (cx * cx + cz * cz) + cy * cy
    dot = lax.dot_general(c, x3, (((1,), (0,)), ((), ())))
    cur = ssc + ssx - 2.0 * dot
    col = lax.broadcasted_iota(jnp.int32, (ts, m), 1)
    big = _F32(3.0e38)
    for kk in range(k):
        mv = jnp.min(cur, axis=1, keepdims=True)
        jk = jnp.min(jnp.where(cur == mv, col, jnp.int32(m)), axis=1,
                     keepdims=True)
        val_ref[:, kk:kk + 1] = mv
        idx_ref[:, kk:kk + 1] = jk
        cur = jnp.where(col == jk, big, cur)


def _knn(cent_r, cand3, k, ts):
    s = cent_r.shape[0]
    m = cand3.shape[1]
    return pl.pallas_call(
        functools.partial(_knn_body, m=m, k=k),
        grid=(s // ts,),
        in_specs=[
            pl.BlockSpec((ts, 3), lambda t: (t, 0)),
            pl.BlockSpec((3, m), lambda t: (0, 0)),
        ],
        out_specs=[
            pl.BlockSpec((ts, k), lambda t: (t, 0)),
            pl.BlockSpec((ts, k), lambda t: (t, 0)),
        ],
        out_shape=[
            jax.ShapeDtypeStruct((s, k), _F32),
            jax.ShapeDtypeStruct((s, k), jnp.int32),
        ],
    )(cent_r, cand3)


def _sc_gather(table, idx):
    b = idx.shape[0]
    d = table.shape[1]
    nw = 32
    bpw = b // nw
    mesh = plsc.VectorSubcoreMesh(core_axis_name="c", subcore_axis_name="s")

    @functools.partial(
        pl.kernel,
        mesh=mesh,
        compiler_params=pltpu.CompilerParams(use_tc_tiling_on_sc=False),
        out_type=jax.ShapeDtypeStruct((b, d), _F32),
        scratch_types=[
            pltpu.VMEM((bpw,), jnp.int32),
            pltpu.VMEM((bpw, d), _F32),
            pltpu.SemaphoreType.DMA,
        ],
    )
    def k(table_hbm, idx_hbm, out_hbm, idx_v, rows_v, sem):
        wid = lax.axis_index("s") * 2 + lax.axis_index("c")
        base = wid * bpw
        pltpu.sync_copy(idx_hbm.at[pl.ds(base, bpw)], idx_v)
        pltpu.async_copy(table_hbm.at[idx_v], rows_v, sem).wait()
        pltpu.sync_copy(rows_v, out_hbm.at[pl.ds(base, bpw)])

    return k(table, idx)


def _sa_body(g_ref, c_ref, w1_ref, w2_ref, o_ref, *, k, dp, d2):
    ts = c_ref.shape[0]
    c = c_ref[...]
    sub = jnp.concatenate([c, jnp.zeros((ts, dp - 3), _F32)], axis=1)
    h = g_ref[...] - sub[None, :, :]
    a = jnp.maximum(
        lax.dot_general(h.reshape(k * ts, dp), w1_ref[...],
                        (((1,), (1,)), ((), ()))) / _BN_DIV, 0.0)
    b = jnp.maximum(
        lax.dot_general(a, w2_ref[...],
                        (((1,), (1,)), ((), ()))) / _BN_DIV, 0.0)
    o_ref[...] = jnp.max(b.reshape(k, ts, d2), axis=0)


def _sa_mlp(g3, cent_r, w1p, w2, ts):
    k, s, dp = g3.shape
    d2 = w2.shape[0]
    return pl.pallas_call(
        functools.partial(_sa_body, k=k, dp=dp, d2=d2),
        grid=(s // ts,),
        in_specs=[
            pl.BlockSpec((k, ts, dp), lambda t: (0, t, 0)),
            pl.BlockSpec((ts, 3), lambda t: (t, 0)),
            pl.BlockSpec(w1p.shape, lambda t: (0, 0)),
            pl.BlockSpec(w2.shape, lambda t: (0, 0)),
        ],
        out_specs=pl.BlockSpec((ts, d2), lambda t: (t, 0)),
        out_shape=jax.ShapeDtypeStruct((s, d2), _F32),
    )(g3, cent_r, w1p, w2)


def _upconv_body(g_ref, c_ref, f1_ref, wg_ref, wf_ref, o_ref, *, k, dp, df):
    ts = c_ref.shape[0]
    c = c_ref[...]
    sub = jnp.concatenate(
        [jnp.zeros((ts, df), _F32), c, jnp.zeros((ts, dp - df - 3), _F32)],
        axis=1)
    h = g_ref[...] - sub[None, :, :]
    a = jnp.maximum(
        lax.dot_general(h.reshape(k * ts, dp), wg_ref[...],
                        (((1,), (1,)), ((), ()))) / _BN_DIV, 0.0)
    hm = jnp.max(a.reshape(k, ts, 32), axis=0)
    h2 = jnp.concatenate([hm, f1_ref[...]], axis=1)
    o = lax.dot_general(h2, wf_ref[...], (((1,), (1,)), ((), ())))
    o_ref[...] = jnp.maximum(o / _BN_DIV, 0.0)


def _upconv(g3, cent_r, f1, wgp, wf, ts):
    k, s, dp = g3.shape
    df = 64
    return pl.pallas_call(
        functools.partial(_upconv_body, k=k, dp=dp, df=df),
        grid=(s // ts,),
        in_specs=[
            pl.BlockSpec((k, ts, dp), lambda t: (0, t, 0)),
            pl.BlockSpec((ts, 3), lambda t: (t, 0)),
            pl.BlockSpec((ts, 32), lambda t: (t, 0)),
            pl.BlockSpec(wgp.shape, lambda t: (0, 0)),
            pl.BlockSpec(wf.shape, lambda t: (0, 0)),
        ],
        out_specs=pl.BlockSpec((ts, 32), lambda t: (t, 0)),
        out_shape=jax.ShapeDtypeStruct((s, 32), _F32),
    )(g3, cent_r, f1, wgp, wf)


def _fp_body(g_ref, w_ref, fea_ref, wp_ref, cw_ref, cb_ref, o_ref, *, dpad):
    ts = w_ref.shape[0]
    g = g_ref[...]
    w = w_ref[...]
    t0 = w[:, 0:1] * g[0]
    t1 = w[:, 1:2] * g[1]
    t2 = w[:, 2:3] * g[2]
    interp = (t0 + t1) + t2
    hcat = jnp.concatenate(
        [interp, fea_ref[...], jnp.zeros((ts, dpad), _F32)], axis=1)
    l0 = jnp.maximum(
        lax.dot_general(hcat, wp_ref[...], (((1,), (1,)), ((), ()))) / _BN_DIV,
        0.0)
    xv = jnp.maximum(l0 / _BN_DIV, 0.0)
    f = lax.dot_general(xv, cw_ref[...], (((1,), (1,)), ((), ()))) + cb_ref[...]
    o_ref[...] = f


def _fp_final(g3, w3, fea, wpp, cw, cb, ts):
    _, s, d = g3.shape
    do = cw.shape[0]
    cb2 = cb.reshape(1, do)
    dpad = wpp.shape[1] - d - fea.shape[1]
    return pl.pallas_call(
        functools.partial(_fp_body, dpad=dpad),
        grid=(s // ts,),
        in_specs=[
            pl.BlockSpec((3, ts, d), lambda t: (0, t, 0)),
            pl.BlockSpec((ts, 3), lambda t: (t, 0)),
            pl.BlockSpec((ts, fea.shape[1]), lambda t: (t, 0)),
            pl.BlockSpec(wpp.shape, lambda t: (0, 0)),
            pl.BlockSpec(cw.shape, lambda t: (0, 0)),
            pl.BlockSpec((1, do), lambda t: (0, 0)),
        ],
        out_specs=pl.BlockSpec((ts, do), lambda t: (t, 0)),
        out_shape=jax.ShapeDtypeStruct((s, do), _F32),
    )(g3, w3, fea, wpp, cw, cb2)


def _padw(w, cols):
    o, c = w.shape
    return jnp.concatenate([w, jnp.zeros((o, cols - c), _F32)], axis=1)


def _sa1_stage(pc, fea, new1r, prm):
    bq1 = _ballq(new1r, pc.T, 1.0, 32, ts=128)
    t1 = jnp.concatenate([pc, fea, jnp.zeros((8192, 3), _F32)], axis=1)
    g1 = _sc_gather(t1, bq1.T.reshape(-1))
    return _sa_mlp(g1.reshape(32, 4096, 16), new1r,
                   _padw(prm['sa1_w1'], 16), prm['sa1_w2'], ts=512)


def _tail_stage(pc, fea, new1r, new2r, l1_f, prm):
    bq2 = _ballq(new2r, new1r.T, 4.0, 32, ts=128)
    t2 = jnp.concatenate([new1r, l1_f, jnp.zeros((4096, 13), _F32)], axis=1)
    g2 = _sc_gather(t2, bq2.T.reshape(-1))
    l2_f = _sa_mlp(g2.reshape(32, 1024, 48), new2r,
                   _padw(prm['sa2_w1'], 48), prm['sa2_w2'], ts=256)

    _, su_idx = _knn(new1r, new2r.T, 8, ts=512)
    tg = jnp.concatenate([l2_f, new2r, jnp.zeros((1024, 13), _F32)], axis=1)
    gg = _sc_gather(tg, su_idx.T.reshape(-1))
    l1_fnew = _upconv(gg.reshape(8, 4096, 80), new1r, l1_f,
                      _padw(prm['su1_wg'], 80), prm['su1_wf'], ts=512)

    d3, fp_idx = _knn(pc, new1r.T, 3, ts=256)
    dist_recip = 1.0 / (d3 + 1e-8)
    w3 = dist_recip / jnp.sum(dist_recip, axis=-1, keepdims=True)
    gf = _sc_gather(l1_fnew, fp_idx.T.reshape(-1))
    f = _fp_final(gf.reshape(3, 8192, 32), w3, fea, _padw(prm['fp_w'], 48),
                  prm['conv2_w'], prm['conv2_b'], ts=1024)
    return jnp.concatenate([pc, f], axis=-1)


def kernel(pc1, fea1, weights1, pc2, fea2, weights2, params):
    pa, fa = pc1[0], fea1[0]
    pb, fb = pc2[0], fea2[0]
    new1a, new1b = _fps2(pa.T, pb.T, pa, pb, 4096)
    l1fa = _sa1_stage(pa, fa, new1a, params)
    l1fb = _sa1_stage(pb, fb, new1b, params)
    new2a, new2b = _fps2(new1a.T, new1b.T, new1a, new1b, 1024)
    sf = _tail_stage(pa, fa, new1a, new2a, l1fa, params)
    tf = _tail_stage(pb, fb, new1b, new2b, l1fb, params)
    return (sf[None], tf[None])
